# baseline (speedup 1.0000x reference)
import functools

import jax
import jax.numpy as jnp
from jax import lax
from jax.experimental import pallas as pl
from jax.experimental.pallas import tpu as pltpu
from jax.experimental.pallas import tpu_sc as plsc

_NC = 2
_NS = 16
_L = 16
_NW = _NC * _NS

_CHUNK = 32768


def _sc_body(emb_hbm, out_hbm, buf, *, scale, n_chunks, words_per_w):
    wid = lax.axis_index("s") * _NC + lax.axis_index("c")
    base = wid * words_per_w

    def chunk_body(ci, carry):
        off = base + ci * _CHUNK
        pltpu.sync_copy(emb_hbm.at[pl.ds(off, _CHUNK)], buf)

        def vec(i, c2):
            sl = pl.ds(i * _L, _L)
            buf[sl] = buf[sl] * scale
            return c2

        lax.fori_loop(0, _CHUNK // _L, vec, 0)
        pltpu.sync_copy(buf, out_hbm.at[pl.ds(off, _CHUNK)])
        return carry

    lax.fori_loop(0, n_chunks, chunk_body, 0)


def kernel(x, emb):
    seq_len = x.shape[1]
    max_len, dim = emb.shape
    scale = float(dim ** -0.5)
    total = seq_len * dim
    words_per_w = total // _NW
    n_chunks = words_per_w // _CHUNK
    emb_flat = emb.reshape(total) if seq_len == max_len else emb[:seq_len].reshape(total)

    body = functools.partial(
        _sc_body, scale=scale, n_chunks=n_chunks, words_per_w=words_per_w
    )
    k = pl.kernel(
        body,
        out_type=jax.ShapeDtypeStruct((total,), jnp.float32),
        mesh=plsc.VectorSubcoreMesh(
            core_axis_name="c", subcore_axis_name="s", num_cores=_NC, num_subcores=_NS
        ),
        scratch_types=[pltpu.VMEM((_CHUNK,), jnp.float32)],
    )
    out = k(emb_flat)
    return out.reshape(seq_len, dim)


# speedup vs baseline: 1.9426x; 1.9426x over previous
import functools

import jax
import jax.numpy as jnp
from jax import lax
from jax.experimental import pallas as pl
from jax.experimental.pallas import tpu as pltpu
from jax.experimental.pallas import tpu_sc as plsc

_NC = 2
_NS = 16
_L = 16
_NW = _NC * _NS

_CHUNK = 32768


def _sc_body(emb_hbm, out_hbm, buf0, buf1, isem0, isem1, osem0, osem1,
             *, scale, n_chunks, words_per_w):
    wid = lax.axis_index("s") * _NC + lax.axis_index("c")
    base = wid * words_per_w
    bufs = (buf0, buf1)
    isems = (isem0, isem1)
    osems = (osem0, osem1)

    def src(ci):
        return emb_hbm.at[pl.ds(base + ci * _CHUNK, _CHUNK)]

    def dst(ci):
        return out_hbm.at[pl.ds(base + ci * _CHUNK, _CHUNK)]

    def compute(buf):
        @plsc.parallel_loop(0, _CHUNK // _L, unroll=8)
        def _(i):
            sl = pl.ds(i * _L, _L)
            buf[sl] = buf[sl] * scale

    pltpu.async_copy(src(0), bufs[0], isems[0])
    for ci in range(n_chunks):
        b = ci % 2
        nb = (ci + 1) % 2
        if ci + 1 < n_chunks:
            if ci >= 1:
                pltpu.make_async_copy(bufs[nb], dst(ci - 1), osems[nb]).wait()
            pltpu.async_copy(src(ci + 1), bufs[nb], isems[nb])
        pltpu.make_async_copy(src(ci), bufs[b], isems[b]).wait()
        compute(bufs[b])
        pltpu.async_copy(bufs[b], dst(ci), osems[b])
    pltpu.make_async_copy(bufs[(n_chunks - 2) % 2], dst(n_chunks - 2),
                          osems[(n_chunks - 2) % 2]).wait()
    pltpu.make_async_copy(bufs[(n_chunks - 1) % 2], dst(n_chunks - 1),
                          osems[(n_chunks - 1) % 2]).wait()


def kernel(x, emb):
    seq_len = x.shape[1]
    max_len, dim = emb.shape
    scale = float(dim ** -0.5)
    total = seq_len * dim
    words_per_w = total // _NW
    n_chunks = words_per_w // _CHUNK
    emb_flat = emb.reshape(total) if seq_len == max_len else emb[:seq_len].reshape(total)

    body = functools.partial(
        _sc_body, scale=scale, n_chunks=n_chunks, words_per_w=words_per_w
    )
    k = pl.kernel(
        body,
        out_type=jax.ShapeDtypeStruct((total,), jnp.float32),
        mesh=plsc.VectorSubcoreMesh(
            core_axis_name="c", subcore_axis_name="s", num_cores=_NC, num_subcores=_NS
        ),
        scratch_types=[
            pltpu.VMEM((_CHUNK,), jnp.float32),
            pltpu.VMEM((_CHUNK,), jnp.float32),
            pltpu.SemaphoreType.DMA,
            pltpu.SemaphoreType.DMA,
            pltpu.SemaphoreType.DMA,
            pltpu.SemaphoreType.DMA,
        ],
    )
    out = k(emb_flat)
    return out.reshape(seq_len, dim)


# speedup vs baseline: 4.5742x; 2.3547x over previous
import functools

import jax
import jax.numpy as jnp
from jax import lax
from jax.experimental import pallas as pl
from jax.experimental.pallas import tpu as pltpu
from jax.experimental.pallas import tpu_sc as plsc

_NC = 2
_NS = 16
_L = 16
_NW = _NC * _NS

_CHUNK_ROWS = 16


def _sc_body(emb_hbm, out_hbm, buf0, buf1, isem0, isem1, osem0, osem1,
             *, scale, dim, n_chunks, rows_per_w):
    wid = lax.axis_index("s") * _NC + lax.axis_index("c")
    base = wid * rows_per_w
    bufs = (buf0, buf1)
    isems = (isem0, isem1)
    osems = (osem0, osem1)

    def src(ci):
        return emb_hbm.at[pl.ds(base + ci * _CHUNK_ROWS, _CHUNK_ROWS), :]

    def dst(ci):
        return out_hbm.at[pl.ds(base + ci * _CHUNK_ROWS, _CHUNK_ROWS), :]

    def start_in(ci):
        pltpu.async_copy(src(ci), bufs[ci % 2], isems[ci % 2])

    def wait_in(ci):
        pltpu.make_async_copy(src(ci), bufs[ci % 2], isems[ci % 2]).wait()

    def start_out(ci):
        pltpu.async_copy(bufs[ci % 2], dst(ci), osems[ci % 2])

    def wait_out(ci):
        pltpu.make_async_copy(bufs[ci % 2], dst(ci), osems[ci % 2]).wait()

    def compute(buf):
        @plsc.parallel_loop(0, dim // _L, unroll=2)
        def _(i):
            sl = pl.ds(i * _L, _L)
            for k in range(_CHUNK_ROWS):
                buf[k, sl] = buf[k, sl] * scale

    start_in(0)
    for ci in range(n_chunks):
        if ci + 1 < n_chunks:
            if ci >= 1:
                wait_out(ci - 1)
            start_in(ci + 1)
        wait_in(ci)
        compute(bufs[ci % 2])
        start_out(ci)
    wait_out(n_chunks - 2)
    wait_out(n_chunks - 1)


def kernel(x, emb):
    seq_len = x.shape[1]
    max_len, dim = emb.shape
    scale = float(dim ** -0.5)
    emb_in = emb if seq_len == max_len else emb[:seq_len]
    rows_per_w = seq_len // _NW
    n_chunks = rows_per_w // _CHUNK_ROWS

    body = functools.partial(
        _sc_body, scale=scale, dim=dim, n_chunks=n_chunks, rows_per_w=rows_per_w
    )
    k = pl.kernel(
        body,
        out_type=jax.ShapeDtypeStruct((seq_len, dim), jnp.float32),
        mesh=plsc.VectorSubcoreMesh(
            core_axis_name="c", subcore_axis_name="s", num_cores=_NC, num_subcores=_NS
        ),
        scratch_types=[
            pltpu.VMEM((_CHUNK_ROWS, dim), jnp.float32),
            pltpu.VMEM((_CHUNK_ROWS, dim), jnp.float32),
            pltpu.SemaphoreType.DMA,
            pltpu.SemaphoreType.DMA,
            pltpu.SemaphoreType.DMA,
            pltpu.SemaphoreType.DMA,
        ],
    )
    return k(emb_in)


# speedup vs baseline: 4.6602x; 1.0188x over previous
import functools

import jax
import jax.numpy as jnp
from jax import lax
from jax.experimental import pallas as pl
from jax.experimental.pallas import tpu as pltpu
from jax.experimental.pallas import tpu_sc as plsc

_NC = 2
_NS = 16
_L = 16
_NW = _NC * _NS

_CHUNK_ROWS = 16
_NBUF = 3


def _sc_body(emb_hbm, out_hbm, *refs, scale, dim, n_chunks, rows_per_w):
    bufs = refs[:_NBUF]
    isems = refs[_NBUF:2 * _NBUF]
    osems = refs[2 * _NBUF:3 * _NBUF]
    wid = lax.axis_index("s") * _NC + lax.axis_index("c")
    base = wid * rows_per_w

    def src(ci):
        return emb_hbm.at[pl.ds(base + ci * _CHUNK_ROWS, _CHUNK_ROWS), :]

    def dst(ci):
        return out_hbm.at[pl.ds(base + ci * _CHUNK_ROWS, _CHUNK_ROWS), :]

    def start_in(ci):
        pltpu.async_copy(src(ci), bufs[ci % _NBUF], isems[ci % _NBUF])

    def wait_in(ci):
        pltpu.make_async_copy(src(ci), bufs[ci % _NBUF], isems[ci % _NBUF]).wait()

    def start_out(ci):
        pltpu.async_copy(bufs[ci % _NBUF], dst(ci), osems[ci % _NBUF])

    def wait_out(ci):
        pltpu.make_async_copy(bufs[ci % _NBUF], dst(ci), osems[ci % _NBUF]).wait()

    def compute(buf):
        @plsc.parallel_loop(0, dim // _L, unroll=2)
        def _(i):
            sl = pl.ds(i * _L, _L)
            for k in range(_CHUNK_ROWS):
                buf[k, sl] = buf[k, sl] * scale

    for ci in range(min(_NBUF - 1, n_chunks)):
        start_in(ci)
    for ci in range(n_chunks):
        pf = ci + _NBUF - 1
        if pf < n_chunks:
            if pf - _NBUF >= 0:
                wait_out(pf - _NBUF)
            start_in(pf)
        wait_in(ci)
        compute(bufs[ci % _NBUF])
        start_out(ci)
    for ci in range(max(0, n_chunks - _NBUF), n_chunks):
        wait_out(ci)


def kernel(x, emb):
    seq_len = x.shape[1]
    max_len, dim = emb.shape
    scale = float(dim ** -0.5)
    emb_in = emb if seq_len == max_len else emb[:seq_len]
    rows_per_w = seq_len // _NW
    n_chunks = rows_per_w // _CHUNK_ROWS

    body = functools.partial(
        _sc_body, scale=scale, dim=dim, n_chunks=n_chunks, rows_per_w=rows_per_w
    )
    k = pl.kernel(
        body,
        out_type=jax.ShapeDtypeStruct((seq_len, dim), jnp.float32),
        mesh=plsc.VectorSubcoreMesh(
            core_axis_name="c", subcore_axis_name="s", num_cores=_NC, num_subcores=_NS
        ),
        scratch_types=(
            [pltpu.VMEM((_CHUNK_ROWS, dim), jnp.float32)] * _NBUF
            + [pltpu.SemaphoreType.DMA] * (2 * _NBUF)
        ),
    )
    return k(emb_in)
